# baseline (speedup 1.0000x reference)
import functools

import jax
import jax.numpy as jnp
from jax import lax
from jax.experimental import pallas as pl
from jax.experimental.pallas import tpu as pltpu
from jax.experimental.pallas import tpu_sc as plsc

_B, _S, _D = 4, 8192, 1024
_NC, _NS = 2, 16
_NW = _NC * _NS
_CH = 8
_LANES = 16
_SPW = _S // _NW
_NCHUNK = _SPW // _CH
_NT = _NCHUNK * _B


def _sc_body(x_hbm, pos_hbm, out_hbm,
             pos_v0, pos_v1, xv0, xv1, xv2, xv3, ov0, ov1, ov2, ov3,
             pi0, pi1, si0, si1, si2, si3, so0, so1, so2, so3):
    pos_bufs, pos_sems = [pos_v0, pos_v1], [pi0, pi1]
    x_bufs = [xv0, xv1, xv2, xv3]
    o_bufs = [ov0, ov1, ov2, ov3]
    in_sems = [si0, si1, si2, si3]
    out_sems = [so0, so1, so2, so3]

    wid = lax.axis_index("s") * _NC + lax.axis_index("c")
    base = wid * _SPW

    def pos_src(ci):
        return pos_hbm.at[pl.ds(base + ci * _CH, _CH)]

    def x_src(ci, b):
        return x_hbm.at[b, pl.ds(base + ci * _CH, _CH)]

    def out_dst(ci, b):
        return out_hbm.at[b, pl.ds(base + ci * _CH, _CH)]

    pltpu.async_copy(pos_src(0), pos_v0, pi0)
    pltpu.async_copy(pos_src(1), pos_v1, pi1)
    pltpu.async_copy(x_src(0, 0), xv0, si0)
    pltpu.async_copy(x_src(0, 1), xv1, si1)
    pltpu.async_copy(x_src(0, 2), xv2, si2)

    @pl.loop(0, _NCHUNK, step=2)
    def _pair(ci0):
        for k in range(2 * _B):
            ci = ci0 + k // _B
            b = k % _B
            slot = k % 4
            pb = k // _B
            t = ci0 * _B + k

            if k % _B == 0:
                pltpu.make_async_copy(pos_src(ci), pos_bufs[pb],
                                      pos_sems[pb]).wait()
            pltpu.make_async_copy(x_src(ci, b), x_bufs[slot],
                                  in_sems[slot]).wait()

            @pl.when(t >= 4)
            def _():
                pltpu.make_async_copy(o_bufs[slot], out_dst(ci, b),
                                      out_sems[slot]).wait()

            xb, ob, pbuf = x_bufs[slot], o_bufs[slot], pos_bufs[pb]

            @plsc.parallel_loop(0, _CH * (_D // _LANES), unroll=8)
            def _elem(i):
                r = i // (_D // _LANES)
                sl = pl.ds((i % (_D // _LANES)) * _LANES, _LANES)
                ob[r, sl] = xb[r, sl] + pbuf[r, sl]

            pltpu.async_copy(ob, out_dst(ci, b), out_sems[slot])

            s2 = (k + 3) % 4
            ci2 = ci0 + (k + 3) // _B
            b2 = (k + 3) % _B

            @pl.when(t + 3 < _NT)
            def _():
                pltpu.async_copy(x_src(ci2, b2), x_bufs[s2], in_sems[s2])

            if k % _B == _B - 1:
                @pl.when(ci + 2 < _NCHUNK)
                def _():
                    pltpu.async_copy(pos_src(ci + 2), pos_bufs[pb],
                                     pos_sems[pb])

    last = _NCHUNK - 1
    pltpu.make_async_copy(ov0, out_dst(last, 0), so0).wait()
    pltpu.make_async_copy(ov1, out_dst(last, 1), so1).wait()
    pltpu.make_async_copy(ov2, out_dst(last, 2), so2).wait()
    pltpu.make_async_copy(ov3, out_dst(last, 3), so3).wait()


_sc_call = functools.partial(
    pl.kernel,
    out_type=jax.ShapeDtypeStruct((_B, _S, _D), jnp.float32),
    mesh=plsc.VectorSubcoreMesh(
        core_axis_name="c", subcore_axis_name="s",
        num_cores=_NC, num_subcores=_NS,
    ),
    scratch_types=(
        [pltpu.VMEM((_CH, _D), jnp.float32)] * 10
        + [pltpu.SemaphoreType.DMA] * 10
    ),
    compiler_params=pltpu.CompilerParams(use_tc_tiling_on_sc=True),
)(_sc_body)


def kernel(x, pos_table):
    B, S, D = x.shape
    return _sc_call(x, pos_table[:S])


# speedup vs baseline: 1.0383x; 1.0383x over previous
import functools

import jax
import jax.numpy as jnp
from jax import lax
from jax.experimental import pallas as pl
from jax.experimental.pallas import tpu as pltpu
from jax.experimental.pallas import tpu_sc as plsc

_B, _S, _D = 4, 8192, 1024
_NC, _NS = 2, 16
_NW = _NC * _NS
_CH = 8
_LANES = 16
_SPW = _S // _NW
_NCHUNK = _SPW // _CH
_NT = _NCHUNK * _B
_NIN = 8
_NOUT = 4


def _sc_body(x_hbm, pos_hbm, out_hbm, *scratch):
    pos_bufs = list(scratch[0:2])
    x_bufs = list(scratch[2:2 + _NIN])
    o_bufs = list(scratch[2 + _NIN:2 + _NIN + _NOUT])
    nb = 2 + _NIN + _NOUT
    pos_sems = list(scratch[nb + 0:nb + 2])
    in_sems = list(scratch[nb + 2:nb + 2 + _NIN])
    out_sems = list(scratch[nb + 2 + _NIN:nb + 2 + _NIN + _NOUT])

    wid = lax.axis_index("s") * _NC + lax.axis_index("c")
    base = wid * _SPW

    def pos_src(ci):
        return pos_hbm.at[pl.ds(base + ci * _CH, _CH)]

    def x_src(ci, b):
        return x_hbm.at[b, pl.ds(base + ci * _CH, _CH)]

    def out_dst(ci, b):
        return out_hbm.at[b, pl.ds(base + ci * _CH, _CH)]

    pltpu.async_copy(pos_src(0), pos_bufs[0], pos_sems[0])
    pltpu.async_copy(pos_src(1), pos_bufs[1], pos_sems[1])
    for t in range(_NIN - 1):
        pltpu.async_copy(x_src(t // _B, t % _B), x_bufs[t % _NIN],
                         in_sems[t % _NIN])

    @pl.loop(0, _NCHUNK, step=2)
    def _pair(ci0):
        for k in range(2 * _B):
            ci = ci0 + k // _B
            b = k % _B
            islot = k % _NIN
            oslot = k % _NOUT
            pb = k // _B
            t = ci0 * _B + k

            if k % _B == 0:
                pltpu.make_async_copy(pos_src(ci), pos_bufs[pb],
                                      pos_sems[pb]).wait()
            pltpu.make_async_copy(x_src(ci, b), x_bufs[islot],
                                  in_sems[islot]).wait()

            @pl.when(t >= _NOUT)
            def _():
                pltpu.make_async_copy(o_bufs[oslot], out_dst(ci, b),
                                      out_sems[oslot]).wait()

            xb, ob, pbuf = x_bufs[islot], o_bufs[oslot], pos_bufs[pb]

            @plsc.parallel_loop(0, _CH * (_D // _LANES), unroll=8)
            def _elem(i):
                r = i // (_D // _LANES)
                sl = pl.ds((i % (_D // _LANES)) * _LANES, _LANES)
                ob[r, sl] = xb[r, sl] + pbuf[r, sl]

            pltpu.async_copy(ob, out_dst(ci, b), out_sems[oslot])

            s2 = (k + _NIN - 1) % _NIN
            ci2 = ci0 + (k + _NIN - 1) // _B
            b2 = (k + _NIN - 1) % _B

            @pl.when(t + _NIN - 1 < _NT)
            def _():
                pltpu.async_copy(x_src(ci2, b2), x_bufs[s2], in_sems[s2])

            if k % _B == _B - 1:
                @pl.when(ci + 2 < _NCHUNK)
                def _():
                    pltpu.async_copy(pos_src(ci + 2), pos_bufs[pb],
                                     pos_sems[pb])

    last = _NCHUNK - 1
    for j in range(_NOUT):
        pltpu.make_async_copy(o_bufs[j], out_dst(last, j),
                              out_sems[j]).wait()


_sc_call = functools.partial(
    pl.kernel,
    out_type=jax.ShapeDtypeStruct((_B, _S, _D), jnp.float32),
    mesh=plsc.VectorSubcoreMesh(
        core_axis_name="c", subcore_axis_name="s",
        num_cores=_NC, num_subcores=_NS,
    ),
    scratch_types=(
        [pltpu.VMEM((_CH, _D), jnp.float32)] * (2 + _NIN + _NOUT)
        + [pltpu.SemaphoreType.DMA] * (2 + _NIN + _NOUT)
    ),
    compiler_params=pltpu.CompilerParams(use_tc_tiling_on_sc=True),
)(_sc_body)


def kernel(x, pos_table):
    B, S, D = x.shape
    return _sc_call(x, pos_table[:S])
